# baseline (speedup 1.0000x reference)
import functools

import jax
import jax.numpy as jnp
from jax import lax
from jax.experimental import pallas as pl
from jax.experimental.pallas import tpu as pltpu
from jax.experimental.pallas import tpu_sc as plsc

_ROWS = 128
_COLS = 32768
_K = 64
_SUB = 8
_LANE = 128
_CPT = 2


@functools.cache
def _make_gather():
    info = plsc.get_sparse_core_info()
    nc, ns = info.num_cores, info.num_subcores

    mesh = plsc.VectorSubcoreMesh(core_axis_name="c", subcore_axis_name="s")

    @functools.partial(
        pl.kernel,
        mesh=mesh,
        out_type=jax.ShapeDtypeStruct((_K, _ROWS), jnp.float32),
        scratch_types=[
            pltpu.VMEM((_K,), jnp.int32),
            pltpu.VMEM((_K,), jnp.int32),
            pltpu.VMEM((_CPT * _ROWS, _LANE), jnp.float32),
            pltpu.VMEM((_CPT, _ROWS), jnp.float32),
            pltpu.VMEM((_SUB, _ROWS), jnp.float32),
            pltpu.VMEM_SHARED((ns, _CPT, _ROWS), jnp.float32),
            pltpu.SemaphoreType.DMA,
            pltpu.SemaphoreType.DMA,
        ],
        compiler_params=pltpu.CompilerParams(needs_layout_passes=False),
    )
    def gather_kernel(
        x_hbm,
        mask_hbm,
        out_hbm,
        mask_v,
        lanes_v,
        blocks_v,
        frag_v,
        vals_v,
        shared,
        sem0,
        sem1,
    ):
        s = lax.axis_index("s")
        c = lax.axis_index("c")
        b = c * 4 + lax.shift_right_logical(s, 2)
        q = lax.bitwise_and(s, 3)
        j0 = b * _SUB + q * _CPT

        pltpu.sync_copy(mask_hbm, mask_v)
        iota = lax.iota(jnp.int32, 16)

        for u in range(_K // 16):
            lanes_v[pl.ds(16 * u, 16)] = lax.bitwise_and(
                mask_v[pl.ds(16 * u, 16)], 127
            )

        def mask_scalar(j):
            pos = jnp.broadcast_to(j, (16,))
            return plsc.load_gather(mask_v, [pos])[0]

        for t in range(_CPT):
            m = mask_scalar(j0 + t)
            mt = pl.multiple_of(
                lax.shift_left(lax.shift_right_logical(m, 7), 7), _LANE
            )
            pltpu.async_copy(
                x_hbm.at[:, pl.ds(mt, _LANE)],
                blocks_v.at[pl.ds(t * _ROWS, _ROWS)],
                sem0 if t == 0 else sem1,
            )

        for t in range(_CPT):
            pltpu.make_async_copy(
                x_hbm.at[:, pl.ds(0, _LANE)],
                blocks_v.at[pl.ds(t * _ROWS, _ROWS)],
                sem0 if t == 0 else sem1,
            ).wait()
            lane = plsc.load_gather(lanes_v, [jnp.broadcast_to(j0 + t, (16,))])

            def pick(k, carry):
                base = pl.multiple_of(16 * k, 16)
                vec = plsc.load_gather(
                    blocks_v, [t * _ROWS + base + iota, lane]
                )
                frag_v[t, pl.ds(base, 16)] = vec
                return carry

            lax.fori_loop(0, _ROWS // 16, pick, 0)

        pltpu.sync_copy(frag_v, shared.at[s])
        plsc.subcore_barrier()

        @pl.when(q == 0)
        def _():
            for g in range(_SUB // _CPT):
                pltpu.sync_copy(
                    shared.at[s + g], vals_v.at[pl.ds(g * _CPT, _CPT)]
                )
            pltpu.sync_copy(
                vals_v, out_hbm.at[pl.ds(pl.multiple_of(b * _SUB, _SUB), _SUB), :]
            )

    return gather_kernel


def kernel(x, mask):
    return _make_gather()(x, mask).T
